# baseline (speedup 1.0000x reference)
import jax
import jax.numpy as jnp
from jax.experimental import pallas as pl

_BN = 256


def _mpnn_block(adj_ref, x_ref, xr_ref, wm_ref, wu_ref, out_ref):
    A = adj_ref[0]
    x = x_ref[0]
    xr = xr_ref[0]
    D = x.shape[-1]
    x_hi = x.astype(jnp.bfloat16)
    x_lo = (x - x_hi.astype(jnp.float32)).astype(jnp.bfloat16)
    dn = (((0,), (0,)), ((), ()))
    S = (jax.lax.dot_general(A, x_hi, dn, preferred_element_type=jnp.float32)
         + jax.lax.dot_general(A, x_lo, dn, preferred_element_type=jnp.float32))
    c = jnp.sum(A.astype(jnp.float32), axis=0)[:, None]
    W1 = wm_ref[:D]
    W2 = wm_ref[D:]
    msg = S @ W1 / jnp.maximum(c, 1.0) + xr @ W2
    msg = jnp.where(c > 0.0, msg, 0.0)
    out = xr @ wu_ref[:D] + msg @ wu_ref[D:]
    out_ref[0] = jnp.maximum(out, 0.0)


def kernel(x, adj, W_msg, W_upd):
    B, N, D = x.shape
    U = W_msg.shape[1]
    adj = adj.astype(jnp.bfloat16)
    return pl.pallas_call(
        _mpnn_block,
        grid=(B, N // _BN),
        in_specs=[
            pl.BlockSpec((1, N, _BN), lambda b, j: (b, 0, j)),
            pl.BlockSpec((1, N, D), lambda b, j: (b, 0, 0)),
            pl.BlockSpec((1, _BN, D), lambda b, j: (b, j, 0)),
            pl.BlockSpec((2 * D, U), lambda b, j: (0, 0)),
            pl.BlockSpec((D + U, U), lambda b, j: (0, 0)),
        ],
        out_specs=pl.BlockSpec((1, _BN, U), lambda b, j: (b, j, 0)),
        out_shape=jax.ShapeDtypeStruct((B, N, U), jnp.float32),
    )(adj, x, x, W_msg, W_upd)


# speedup vs baseline: 1.1588x; 1.1588x over previous
import jax
import jax.numpy as jnp
from jax.experimental import pallas as pl


def _mpnn_block(adj_ref, xa_ref, x_ref, wm_ref, wu_ref, out_ref):
    A = adj_ref[0]
    xa = xa_ref[0]
    x = x_ref[0]
    D = x.shape[-1]
    dn = (((0,), (0,)), ((), ()))
    Sa = jax.lax.dot_general(A, xa, dn, preferred_element_type=jnp.float32)
    S = Sa[:, :D]
    c = Sa[:, D:D + 1]
    rinv = jnp.where(c > 0.0, 1.0 / jnp.maximum(c, 1.0), 0.0)
    pos = jnp.where(c > 0.0, 1.0, 0.0)
    msg = (S @ wm_ref[:D]) * rinv + (x @ wm_ref[D:]) * pos
    out = x @ wu_ref[:D] + msg @ wu_ref[D:]
    out_ref[0] = jnp.maximum(out, 0.0)


def kernel(x, adj, W_msg, W_upd):
    B, N, D = x.shape
    U = W_msg.shape[1]
    adj = adj.astype(jnp.bfloat16)
    xa = jnp.concatenate(
        [x.astype(jnp.bfloat16), jnp.ones((B, N, 1), jnp.bfloat16)], axis=-1)
    return pl.pallas_call(
        _mpnn_block,
        grid=(B,),
        in_specs=[
            pl.BlockSpec((1, N, N), lambda b: (b, 0, 0)),
            pl.BlockSpec((1, N, D + 1), lambda b: (b, 0, 0)),
            pl.BlockSpec((1, N, D), lambda b: (b, 0, 0)),
            pl.BlockSpec((2 * D, U), lambda b: (0, 0)),
            pl.BlockSpec((D + U, U), lambda b: (0, 0)),
        ],
        out_specs=pl.BlockSpec((1, N, U), lambda b: (b, 0, 0)),
        out_shape=jax.ShapeDtypeStruct((B, N, U), jnp.float32),
    )(adj, xa, x, W_msg, W_upd)
